# Initial kernel scaffold; baseline (speedup 1.0000x reference)
import jax
import jax.numpy as jnp
from jax.experimental import pallas as pl


def kernel(x_a, x_b, W_root_a, b_root_a, W_root_b, b_root_b, W_rel_ab, W_rel_ba, edge_index_ab, edge_index_ba):
    raise NotImplementedError("write your pallas kernel here")


# speedup vs baseline: 2.8087x; 2.8087x over previous
import functools

import jax
import jax.numpy as jnp
from jax import lax
from jax.experimental import pallas as pl
from jax.experimental.pallas import tpu as pltpu
from jax.experimental.pallas import tpu_sc as plsc

N = 10000
E = 160000
D = 256
DH = 128
NS = 16
K = 128
EPT = E // NS
EPT_PAD = 10240
CHUNKS = EPT_PAD // K
PAIRS = CHUNKS // 2
ROWN = 640
STG = 128
BR = 1000


def _tc_prepare(x_a, x_b, Wra, bra, Wrb, brb, Wab, Wba):
    dn = (((1,), (1,)), ((), ()))

    def body(xa_ref, xb_ref, wra_ref, bra_ref, wrb_ref, brb_ref,
             wab_ref, wba_ref,
             ra_ref, rb_ref, yal_ref, yah_ref, ybl_ref, ybh_ref):
        xa = xa_ref[...]
        xb = xb_ref[...]
        ra_ref[...] = lax.dot_general(
            xa, wra_ref[...], dn, preferred_element_type=jnp.float32
        ) + bra_ref[...]
        rb_ref[...] = lax.dot_general(
            xb, wrb_ref[...], dn, preferred_element_type=jnp.float32
        ) + brb_ref[...]
        ya = lax.dot_general(xa, wab_ref[...], dn,
                             preferred_element_type=jnp.float32)
        yb = lax.dot_general(xb, wba_ref[...], dn,
                             preferred_element_type=jnp.float32)
        yal_ref[...] = ya[:, :DH]
        yah_ref[...] = ya[:, DH:]
        ybl_ref[...] = yb[:, :DH]
        ybh_ref[...] = yb[:, DH:]

    row_spec = pl.BlockSpec((BR, D), lambda i: (i, 0))
    half_spec = pl.BlockSpec((BR, DH), lambda i: (i, 0))
    w_spec = pl.BlockSpec((D, D), lambda i: (0, 0))
    b_spec = pl.BlockSpec((1, D), lambda i: (0, 0))
    return pl.pallas_call(
        body,
        grid=(N // BR,),
        in_specs=[row_spec, row_spec, w_spec, b_spec, w_spec, b_spec,
                  w_spec, w_spec],
        out_specs=[row_spec, row_spec, half_spec, half_spec, half_spec,
                   half_spec],
        out_shape=[
            jax.ShapeDtypeStruct((N, D), jnp.float32),
            jax.ShapeDtypeStruct((N, D), jnp.float32),
            jax.ShapeDtypeStruct((N, DH), jnp.float32),
            jax.ShapeDtypeStruct((N, DH), jnp.float32),
            jax.ShapeDtypeStruct((N, DH), jnp.float32),
            jax.ShapeDtypeStruct((N, DH), jnp.float32),
        ],
    )(x_a, x_b, Wra, bra.reshape(1, D), Wrb, brb.reshape(1, D), Wab, Wba)


def _sc_segment_sums(yal, yah, ybl, ybh, sab, dab, sba, dba,
                     ones_h, z128_h):
    mesh = plsc.VectorSubcoreMesh(core_axis_name="c", subcore_axis_name="s")

    @functools.partial(
        pl.kernel,
        mesh=mesh,
        out_type=[
            jax.ShapeDtypeStruct((N, DH), jnp.float32),
            jax.ShapeDtypeStruct((N, DH), jnp.float32),
            jax.ShapeDtypeStruct((N, DH), jnp.float32),
            jax.ShapeDtypeStruct((N, DH), jnp.float32),
            jax.ShapeDtypeStruct((N, DH), jnp.float32),
            jax.ShapeDtypeStruct((N, DH), jnp.float32),
        ],
        scratch_types=[
            pltpu.VMEM((2, K), jnp.int32),
            pltpu.VMEM((2, K), jnp.int32),
            pltpu.VMEM((K, DH), jnp.float32),
            pltpu.VMEM((K, DH), jnp.float32),
            pltpu.VMEM((STG, DH), jnp.float32),
            pltpu.VMEM((STG,), jnp.int32),
            pltpu.VMEM_SHARED((N + 8, DH), jnp.float32),
            pltpu.SemaphoreType.DMA,
            pltpu.SemaphoreType.DMA,
            pltpu.SemaphoreType.DMA,
            pltpu.SemaphoreType.DMA,
            pltpu.SemaphoreType.DMA,
            pltpu.SemaphoreType.DMA,
            pltpu.SemaphoreType.DMA,
        ],
    )
    def k(yal_h, yah_h, ybl_h, ybh_h, sab_h, dab_h, sba_h, dba_h,
          ones_hbm, z128_hbm,
          o_bl, o_bh, o_bc, o_al, o_ah, o_ac,
          idx_s, idx_d, rows0, rows1, zstg, ridx, acc,
          sem_i0, sem_i1, sem_g0, sem_g1, sem_s0, sem_s1, sem_z):
        c = lax.axis_index("c")
        s = lax.axis_index("s")
        base_e = pl.multiple_of(s * EPT_PAD, 8)
        base_r = s * ROWN
        lane = lax.iota(jnp.int32, 16)
        rows = (rows0, rows1)
        sem_i = (sem_i0, sem_i1)
        sem_g = (sem_g0, sem_g1)
        sem_s = (sem_s0, sem_s1)

        def fill_ridx(j):
            for jj in range(STG // 16):
                v = base_r + (j * STG + jj * 16) + lane
                ridx[pl.ds(jj * 16, 16)] = jnp.minimum(v, N - 1)

        def run_rel(tbl_l, tbl_h, src, dst, out_l, out_h, out_c):
            for p in range(3):
                tbl = (tbl_l, tbl_h, None)[p]
                out = (out_l, out_h, out_c)[p]
                pltpu.sync_copy(z128_hbm, zstg)
                if p == 2:
                    pltpu.sync_copy(ones_hbm, rows0)
                    pltpu.sync_copy(ones_hbm, rows1)
                for j in range(ROWN // STG):
                    fill_ridx(j)
                    pltpu.async_copy(zstg, acc.at[ridx], sem_z).wait()
                if p == 0:
                    @pl.when(s == 0)
                    def _():
                        for jj in range(STG // 16):
                            ridx[pl.ds(jj * 16, 16)] = jnp.minimum(
                                N + lane, N + 7)
                        pltpu.async_copy(zstg, acc.at[ridx],
                                         sem_z).wait()
                plsc.subcore_barrier()

                def pair(i2, carry):
                    hs = []
                    for b in range(2):
                        off = pl.multiple_of(
                            base_e + (2 * i2 + b) * K, 8)
                        hd = pltpu.async_copy(
                            dst.at[pl.ds(off, K)], idx_d.at[b],
                            sem_i[b])
                        if p < 2:
                            hsrc = pltpu.async_copy(
                                src.at[pl.ds(off, K)], idx_s.at[b],
                                sem_i[b])
                        else:
                            hsrc = None
                        hs.append((hd, hsrc))
                    gs = []
                    for b in range(2):
                        hd, hsrc = hs[b]
                        hd.wait()
                        if p < 2:
                            hsrc.wait()
                            gs.append(pltpu.async_copy(
                                tbl.at[idx_s.at[b]], rows[b], sem_g[b]))
                    ss = []
                    for b in range(2):
                        if p < 2:
                            gs[b].wait()
                        ss.append(pltpu.async_copy(
                            rows[b], acc.at[idx_d.at[b]], sem_s[b],
                            add=True))
                    for b in range(2):
                        ss[b].wait()
                    return carry

                lax.fori_loop(0, PAIRS, pair, 0)
                plsc.subcore_barrier()
                for j in range(ROWN // STG):
                    fill_ridx(j)
                    pltpu.async_copy(acc.at[ridx], zstg, sem_z).wait()
                    pltpu.async_copy(zstg, out.at[ridx], sem_z).wait()
                plsc.subcore_barrier()

        @pl.when(c == 0)
        def _():
            run_rel(yal_h, yah_h, sab_h, dab_h, o_bl, o_bh, o_bc)

        @pl.when(c == 1)
        def _():
            run_rel(ybl_h, ybh_h, sba_h, dba_h, o_al, o_ah, o_ac)

    return k(yal, yah, ybl, ybh, sab, dab, sba, dba, ones_h, z128_h)


def _tc_combine(ra, rb, s_al, s_ah, c_a, s_bl, s_bh, c_b):
    def body(ra_ref, rb_ref, al_ref, ah_ref, ca_ref, bl_ref, bh_ref, cb_ref,
             oa_ref, ob_ref):
        inv_a = 1.0 / jnp.maximum(ca_ref[:, 0:1], 1.0)
        inv_b = 1.0 / jnp.maximum(cb_ref[:, 0:1], 1.0)
        oa_ref[:, :DH] = ra_ref[:, :DH] + al_ref[...] * inv_a
        oa_ref[:, DH:] = ra_ref[:, DH:] + ah_ref[...] * inv_a
        ob_ref[:, :DH] = rb_ref[:, :DH] + bl_ref[...] * inv_b
        ob_ref[:, DH:] = rb_ref[:, DH:] + bh_ref[...] * inv_b

    row_spec = pl.BlockSpec((BR, D), lambda i: (i, 0))
    half_spec = pl.BlockSpec((BR, DH), lambda i: (i, 0))
    return pl.pallas_call(
        body,
        grid=(N // BR,),
        in_specs=[row_spec, row_spec, half_spec, half_spec, half_spec,
                  half_spec, half_spec, half_spec],
        out_specs=[row_spec, row_spec],
        out_shape=[
            jax.ShapeDtypeStruct((N, D), jnp.float32),
            jax.ShapeDtypeStruct((N, D), jnp.float32),
        ],
    )(ra, rb, s_al, s_ah, c_a, s_bl, s_bh, c_b)


def _pad_edges(e):
    src = e[0].reshape(NS, EPT)
    dst = e[1].reshape(NS, EPT)
    pad = EPT_PAD - EPT
    src = jnp.concatenate(
        [src, jnp.zeros((NS, pad), jnp.int32)], axis=1).reshape(-1)
    dst = jnp.concatenate(
        [dst, jnp.full((NS, pad), N, jnp.int32)], axis=1).reshape(-1)
    return src, dst


def kernel(x_a, x_b, W_root_a, b_root_a, W_root_b, b_root_b,
           W_rel_ab, W_rel_ba, edge_index_ab, edge_index_ba):
    sab, dab = _pad_edges(edge_index_ab.astype(jnp.int32))
    sba, dba = _pad_edges(edge_index_ba.astype(jnp.int32))

    ra, rb, yal, yah, ybl, ybh = _tc_prepare(
        x_a, x_b, W_root_a, b_root_a, W_root_b, b_root_b, W_rel_ab, W_rel_ba)

    ones_h = jnp.ones((K, DH), jnp.float32)
    z128_h = jnp.zeros((STG, DH), jnp.float32)

    s_bl, s_bh, c_b, s_al, s_ah, c_a = _sc_segment_sums(
        yal, yah, ybl, ybh, sab, dab, sba, dba, ones_h, z128_h)

    out_a, out_b = _tc_combine(ra, rb, s_al, s_ah, c_a, s_bl, s_bh, c_b)
    return (out_a, out_b)


# speedup vs baseline: 2.8571x; 1.0172x over previous
import functools

import jax
import jax.numpy as jnp
from jax import lax
from jax.experimental import pallas as pl
from jax.experimental.pallas import tpu as pltpu
from jax.experimental.pallas import tpu_sc as plsc

N = 10000
E = 160000
D = 256
DH = 128
NS = 16
K = 128
EPT = E // NS
EPT_PAD = 10240
CHUNKS = EPT_PAD // K
PAIRS = CHUNKS // 2
ROWN = 640
STG = 128
BR = 1000


def _tc_prepare(x_a, x_b, Wra, bra, Wrb, brb, Wab, Wba):
    dn = (((1,), (1,)), ((), ()))

    def body(xa_ref, xb_ref, wra_ref, bra_ref, wrb_ref, brb_ref,
             wab_ref, wba_ref,
             ra_ref, rb_ref, yal_ref, yah_ref, ybl_ref, ybh_ref):
        xa = xa_ref[...]
        xb = xb_ref[...]
        ra_ref[...] = lax.dot_general(
            xa, wra_ref[...], dn, preferred_element_type=jnp.float32
        ) + bra_ref[...]
        rb_ref[...] = lax.dot_general(
            xb, wrb_ref[...], dn, preferred_element_type=jnp.float32
        ) + brb_ref[...]
        ya = lax.dot_general(xa, wab_ref[...], dn,
                             preferred_element_type=jnp.float32)
        yb = lax.dot_general(xb, wba_ref[...], dn,
                             preferred_element_type=jnp.float32)
        yal_ref[...] = ya[:, :DH]
        yah_ref[...] = ya[:, DH:]
        ybl_ref[...] = yb[:, :DH]
        ybh_ref[...] = yb[:, DH:]

    row_spec = pl.BlockSpec((BR, D), lambda i: (i, 0))
    half_spec = pl.BlockSpec((BR, DH), lambda i: (i, 0))
    w_spec = pl.BlockSpec((D, D), lambda i: (0, 0))
    b_spec = pl.BlockSpec((1, D), lambda i: (0, 0))
    return pl.pallas_call(
        body,
        grid=(N // BR,),
        in_specs=[row_spec, row_spec, w_spec, b_spec, w_spec, b_spec,
                  w_spec, w_spec],
        out_specs=[row_spec, row_spec, half_spec, half_spec, half_spec,
                   half_spec],
        out_shape=[
            jax.ShapeDtypeStruct((N, D), jnp.float32),
            jax.ShapeDtypeStruct((N, D), jnp.float32),
            jax.ShapeDtypeStruct((N, DH), jnp.float32),
            jax.ShapeDtypeStruct((N, DH), jnp.float32),
            jax.ShapeDtypeStruct((N, DH), jnp.float32),
            jax.ShapeDtypeStruct((N, DH), jnp.float32),
        ],
    )(x_a, x_b, Wra, bra.reshape(1, D), Wrb, brb.reshape(1, D), Wab, Wba)


def _sc_segment_sums(yal, yah, ybl, ybh, sab, dab, sba, dba,
                     ones_h, z128_h):
    mesh = plsc.VectorSubcoreMesh(core_axis_name="c", subcore_axis_name="s")

    @functools.partial(
        pl.kernel,
        mesh=mesh,
        out_type=[
            jax.ShapeDtypeStruct((N, DH), jnp.float32),
            jax.ShapeDtypeStruct((N, DH), jnp.float32),
            jax.ShapeDtypeStruct((N, DH), jnp.float32),
            jax.ShapeDtypeStruct((N, DH), jnp.float32),
            jax.ShapeDtypeStruct((N, DH), jnp.float32),
            jax.ShapeDtypeStruct((N, DH), jnp.float32),
        ],
        scratch_types=[
            pltpu.VMEM((2, K), jnp.int32),
            pltpu.VMEM((2, K), jnp.int32),
            pltpu.VMEM((K, DH), jnp.float32),
            pltpu.VMEM((K, DH), jnp.float32),
            pltpu.VMEM((STG, DH), jnp.float32),
            pltpu.VMEM((STG,), jnp.int32),
            pltpu.VMEM_SHARED((N + 8, DH), jnp.float32),
            pltpu.SemaphoreType.DMA,
            pltpu.SemaphoreType.DMA,
            pltpu.SemaphoreType.DMA,
            pltpu.SemaphoreType.DMA,
            pltpu.SemaphoreType.DMA,
            pltpu.SemaphoreType.DMA,
            pltpu.SemaphoreType.DMA,
        ],
    )
    def k(yal_h, yah_h, ybl_h, ybh_h, sab_h, dab_h, sba_h, dba_h,
          ones_hbm, z128_hbm,
          o_bl, o_bh, o_bc, o_al, o_ah, o_ac,
          idx_s, idx_d, rows0, rows1, zstg, ridx, acc,
          sem_i0, sem_i1, sem_g0, sem_g1, sem_s0, sem_s1, sem_z):
        c = lax.axis_index("c")
        s = lax.axis_index("s")
        base_e = pl.multiple_of(s * EPT_PAD, 8)
        base_r = s * ROWN
        lane = lax.iota(jnp.int32, 16)
        rows = (rows0, rows1)
        sem_i = (sem_i0, sem_i1)
        sem_g = (sem_g0, sem_g1)
        sem_s = (sem_s0, sem_s1)

        def fill_ridx(j):
            for jj in range(STG // 16):
                v = base_r + (j * STG + jj * 16) + lane
                ridx[pl.ds(jj * 16, 16)] = jnp.minimum(v, N - 1)

        def run_rel(tbl_l, tbl_h, src, dst, out_l, out_h, out_c):
            for p in range(3):
                tbl = (tbl_l, tbl_h, None)[p]
                out = (out_l, out_h, out_c)[p]
                pltpu.sync_copy(z128_hbm, zstg)
                if p == 2:
                    pltpu.sync_copy(ones_hbm, rows0)
                    pltpu.sync_copy(ones_hbm, rows1)
                for j in range(ROWN // STG):
                    fill_ridx(j)
                    pltpu.async_copy(zstg, acc.at[ridx], sem_z).wait()
                if p == 0:
                    @pl.when(s == 0)
                    def _():
                        for jj in range(STG // 16):
                            ridx[pl.ds(jj * 16, 16)] = jnp.minimum(
                                N + lane, N + 7)
                        pltpu.async_copy(zstg, acc.at[ridx],
                                         sem_z).wait()
                plsc.subcore_barrier()

                def pair(i2, carry):
                    @pl.when(i2 > 0)
                    def _():
                        for b in range(2):
                            pltpu.make_async_copy(
                                z128_hbm, rows[b], sem_s[b]).wait()

                    hs = []
                    for b in range(2):
                        off = pl.multiple_of(
                            base_e + (2 * i2 + b) * K, 8)
                        hd = pltpu.async_copy(
                            dst.at[pl.ds(off, K)], idx_d.at[b],
                            sem_i[b])
                        if p < 2:
                            hsrc = pltpu.async_copy(
                                src.at[pl.ds(off, K)], idx_s.at[b],
                                sem_i[b])
                        else:
                            hsrc = None
                        hs.append((hd, hsrc))
                    gs = []
                    for b in range(2):
                        hd, hsrc = hs[b]
                        hd.wait()
                        if p < 2:
                            hsrc.wait()
                            gs.append(pltpu.async_copy(
                                tbl.at[idx_s.at[b]], rows[b], sem_g[b]))
                    for b in range(2):
                        if p < 2:
                            gs[b].wait()
                        pltpu.async_copy(
                            rows[b], acc.at[idx_d.at[b]], sem_s[b],
                            add=True)
                    return carry

                lax.fori_loop(0, PAIRS, pair, 0)
                for b in range(2):
                    pltpu.make_async_copy(
                        z128_hbm, rows[b], sem_s[b]).wait()
                plsc.subcore_barrier()
                for j in range(ROWN // STG):
                    fill_ridx(j)
                    pltpu.async_copy(acc.at[ridx], zstg, sem_z).wait()
                    pltpu.async_copy(zstg, out.at[ridx], sem_z).wait()
                plsc.subcore_barrier()

        @pl.when(c == 0)
        def _():
            run_rel(yal_h, yah_h, sab_h, dab_h, o_bl, o_bh, o_bc)

        @pl.when(c == 1)
        def _():
            run_rel(ybl_h, ybh_h, sba_h, dba_h, o_al, o_ah, o_ac)

    return k(yal, yah, ybl, ybh, sab, dab, sba, dba, ones_h, z128_h)


def _tc_combine(ra, rb, s_al, s_ah, c_a, s_bl, s_bh, c_b):
    def body(ra_ref, rb_ref, al_ref, ah_ref, ca_ref, bl_ref, bh_ref, cb_ref,
             oa_ref, ob_ref):
        inv_a = 1.0 / jnp.maximum(ca_ref[:, 0:1], 1.0)
        inv_b = 1.0 / jnp.maximum(cb_ref[:, 0:1], 1.0)
        oa_ref[:, :DH] = ra_ref[:, :DH] + al_ref[...] * inv_a
        oa_ref[:, DH:] = ra_ref[:, DH:] + ah_ref[...] * inv_a
        ob_ref[:, :DH] = rb_ref[:, :DH] + bl_ref[...] * inv_b
        ob_ref[:, DH:] = rb_ref[:, DH:] + bh_ref[...] * inv_b

    row_spec = pl.BlockSpec((BR, D), lambda i: (i, 0))
    half_spec = pl.BlockSpec((BR, DH), lambda i: (i, 0))
    return pl.pallas_call(
        body,
        grid=(N // BR,),
        in_specs=[row_spec, row_spec, half_spec, half_spec, half_spec,
                  half_spec, half_spec, half_spec],
        out_specs=[row_spec, row_spec],
        out_shape=[
            jax.ShapeDtypeStruct((N, D), jnp.float32),
            jax.ShapeDtypeStruct((N, D), jnp.float32),
        ],
    )(ra, rb, s_al, s_ah, c_a, s_bl, s_bh, c_b)


def _pad_edges(e):
    src = e[0].reshape(NS, EPT)
    dst = e[1].reshape(NS, EPT)
    pad = EPT_PAD - EPT
    src = jnp.concatenate(
        [src, jnp.zeros((NS, pad), jnp.int32)], axis=1).reshape(-1)
    dst = jnp.concatenate(
        [dst, jnp.full((NS, pad), N, jnp.int32)], axis=1).reshape(-1)
    return src, dst


def kernel(x_a, x_b, W_root_a, b_root_a, W_root_b, b_root_b,
           W_rel_ab, W_rel_ba, edge_index_ab, edge_index_ba):
    sab, dab = _pad_edges(edge_index_ab.astype(jnp.int32))
    sba, dba = _pad_edges(edge_index_ba.astype(jnp.int32))

    ra, rb, yal, yah, ybl, ybh = _tc_prepare(
        x_a, x_b, W_root_a, b_root_a, W_root_b, b_root_b, W_rel_ab, W_rel_ba)

    ones_h = jnp.ones((K, DH), jnp.float32)
    z128_h = jnp.zeros((STG, DH), jnp.float32)

    s_bl, s_bh, c_b, s_al, s_ah, c_a = _sc_segment_sums(
        yal, yah, ybl, ybh, sab, dab, sba, dba, ones_h, z128_h)

    out_a, out_b = _tc_combine(ra, rb, s_al, s_ah, c_a, s_bl, s_bh, c_b)
    return (out_a, out_b)
